# Initial kernel scaffold; baseline (speedup 1.0000x reference)
import jax
import jax.numpy as jnp
from jax.experimental import pallas as pl


def kernel(x, edge_index, edge_weight, W0, W1, W_lin):
    raise NotImplementedError("write your pallas kernel here")


# speedup vs baseline: 6.3545x; 6.3545x over previous
import functools

import jax
import jax.numpy as jnp
from jax import lax
from jax.experimental import pallas as pl
from jax.experimental.pallas import tpu as pltpu
from jax.experimental.pallas import tpu_sc as plsc

N = 10000
E = 320000
D = 128
H = 128
GROUP = 10

NC = 2
NS = 16
NW = NC * NS
WIN = 128
NWIN = 80
CHW = 16
NCH = NWIN // CHW
EPW = WIN * NWIN
EPAD = NW * EPW
NP = 10240
ROWS_PER_TILE = NP // NS

_mesh = functools.partial(
    plsc.VectorSubcoreMesh, core_axis_name="c", subcore_axis_name="s",
    num_cores=NC, num_subcores=NS)


def _deg_body(col_hbm, w_hbm, deg_out, col_v, w_v, zero_v, deg_sp):
    cid = lax.axis_index("c")
    sid = lax.axis_index("s")
    wid = cid * NS + sid

    pltpu.sync_copy(col_hbm.at[wid], col_v)
    pltpu.sync_copy(w_hbm.at[wid], w_v)

    @pl.when(sid == 0)
    def _():
        z16 = jnp.zeros((16,), jnp.float32)

        def zb(i, c):
            zero_v[pl.ds(i * 16, 16)] = z16
            return c
        lax.fori_loop(0, NP // 16, zb, 0)
        pltpu.sync_copy(zero_v, deg_sp)

    plsc.subcore_barrier()

    def win(wi, c):
        pltpu.sync_copy(w_v.at[wi], deg_sp.at[col_v.at[wi]], add=True)
        return c
    lax.fori_loop(0, NWIN, win, 0)

    plsc.subcore_barrier()

    @pl.when(sid == 0)
    def _():
        pltpu.sync_copy(deg_sp, deg_out.at[cid])


def _sc_degree(col3, w3):
    k = pl.kernel(
        _deg_body,
        out_type=jax.ShapeDtypeStruct((NC, NP), jnp.float32),
        mesh=_mesh(),
        compiler_params=pltpu.CompilerParams(needs_layout_passes=False),
        scratch_types=[
            pltpu.VMEM((NWIN, WIN), jnp.int32),
            pltpu.VMEM((NWIN, WIN), jnp.float32),
            pltpu.VMEM((NP,), jnp.float32),
            pltpu.VMEM_SHARED((NP,), jnp.float32),
        ],
    )
    return k(col3, w3)


def _agg_body(xw_hbm, row_hbm, col_hbm, w_hbm, dis_hbm, out_hbm,
              row_v, col_v, w_v, dis_v, msg_v, acc_sp, sem):
    cid = lax.axis_index("c")
    sid = lax.axis_index("s")
    wid = cid * NS + sid

    pltpu.sync_copy(dis_hbm, dis_v)

    z16 = jnp.zeros((16,), jnp.float32)

    def zb(i, c):
        for g in range(8):
            msg_v[i, pl.ds(g * 16, 16)] = z16
        return c
    lax.fori_loop(0, WIN, zb, 0)
    for j in range(ROWS_PER_TILE // WIN):
        pltpu.sync_copy(msg_v, acc_sp.at[pl.ds(sid * ROWS_PER_TILE + j * WIN, WIN)])

    plsc.subcore_barrier()

    def chunk(ci, c0):
        pltpu.sync_copy(row_hbm.at[wid, pl.ds(ci * CHW, CHW)], row_v)
        pltpu.sync_copy(col_hbm.at[wid, pl.ds(ci * CHW, CHW)], col_v)
        pltpu.sync_copy(w_hbm.at[wid, pl.ds(ci * CHW, CHW)], w_v)

        def win(wi, c):
            pltpu.async_copy(xw_hbm.at[row_v.at[wi]], msg_v, sem).wait()

            def scale(g, c2):
                r16 = row_v[wi, pl.ds(g * 16, 16)]
                c16 = col_v[wi, pl.ds(g * 16, 16)]
                w16 = w_v[wi, pl.ds(g * 16, 16)]
                nm16 = plsc.load_gather(dis_v, [r16]) * w16 * plsc.load_gather(dis_v, [c16])
                for l in range(16):
                    sv = jnp.full((16,), nm16[l], jnp.float32)
                    e = g * 16 + l
                    for ch in range(8):
                        msg_v[e, pl.ds(ch * 16, 16)] = msg_v[e, pl.ds(ch * 16, 16)] * sv
                return c2
            lax.fori_loop(0, 8, scale, 0)

            pltpu.sync_copy(msg_v, acc_sp.at[col_v.at[wi]], add=True)
            return c
        lax.fori_loop(0, CHW, win, 0)
        return c0
    lax.fori_loop(0, NCH, chunk, 0)

    plsc.subcore_barrier()
    pltpu.sync_copy(acc_sp.at[pl.ds(sid * ROWS_PER_TILE, ROWS_PER_TILE)],
                    out_hbm.at[cid, pl.ds(sid * ROWS_PER_TILE, ROWS_PER_TILE)])


def _sc_aggregate(xw, row3, col3, w3, dis):
    k = pl.kernel(
        _agg_body,
        out_type=jax.ShapeDtypeStruct((NC, NP, H), jnp.float32),
        mesh=_mesh(),
        compiler_params=pltpu.CompilerParams(needs_layout_passes=False),
        scratch_types=[
            pltpu.VMEM((CHW, WIN), jnp.int32),
            pltpu.VMEM((CHW, WIN), jnp.int32),
            pltpu.VMEM((CHW, WIN), jnp.float32),
            pltpu.VMEM((N,), jnp.float32),
            pltpu.VMEM((WIN, H), jnp.float32),
            pltpu.VMEM_SHARED((NP, H), jnp.float32),
            pltpu.SemaphoreType.DMA,
        ],
    )
    return k(xw, row3, col3, w3, dis)


def _dis_body(dp_ref, out_ref):
    deg = dp_ref[0] + dp_ref[1]
    out_ref[...] = jnp.where(
        deg > 0, lax.rsqrt(jnp.maximum(deg, 1e-12)), 0.0)


def _tc_dis(deg_parts):
    npad = 80 * 128
    dp = jnp.pad(deg_parts, ((0, 0), (0, npad - N))).reshape(NC, 80, 128)
    out = pl.pallas_call(
        _dis_body,
        out_shape=jax.ShapeDtypeStruct((80, 128), jnp.float32),
    )(dp)
    return out.reshape(npad)[:N]


def _mm_body(x_ref, w_ref, o_ref):
    o_ref[...] = jnp.dot(x_ref[...], w_ref[...],
                         preferred_element_type=jnp.float32)


def _tc_matmul(x, w):
    m = x.shape[0]
    bm = 400
    return pl.pallas_call(
        _mm_body,
        grid=(m // bm,),
        in_specs=[pl.BlockSpec((bm, x.shape[1]), lambda i: (i, 0)),
                  pl.BlockSpec(w.shape, lambda i: (0, 0))],
        out_specs=pl.BlockSpec((bm, w.shape[1]), lambda i: (i, 0)),
        out_shape=jax.ShapeDtypeStruct((m, w.shape[1]), jnp.float32),
    )(x, w)


def _mid_body(a_ref, b_ref, w_ref, o_ref):
    h = jnp.maximum(a_ref[...] + b_ref[...], 0.0)
    o_ref[...] = jnp.dot(h, w_ref[...], preferred_element_type=jnp.float32)


def _tc_relu_matmul(a, b, w):
    m = a.shape[0]
    bm = 400
    return pl.pallas_call(
        _mid_body,
        grid=(m // bm,),
        in_specs=[pl.BlockSpec((bm, a.shape[1]), lambda i: (i, 0)),
                  pl.BlockSpec((bm, a.shape[1]), lambda i: (i, 0)),
                  pl.BlockSpec(w.shape, lambda i: (0, 0))],
        out_specs=pl.BlockSpec((bm, w.shape[1]), lambda i: (i, 0)),
        out_shape=jax.ShapeDtypeStruct((m, w.shape[1]), jnp.float32),
    )(a, b, w)


def _head_body(a_ref, b_ref, w_ref, o_ref):
    h = jnp.maximum(a_ref[...] + b_ref[...], 0.0)
    p = jnp.sum(h * w_ref[...], axis=1, keepdims=True)
    o_ref[...] = jnp.maximum(p, 0.0)


def _tc_head(c0, c1, w_row):
    m = c0.shape[0]
    return pl.pallas_call(
        _head_body,
        out_shape=jax.ShapeDtypeStruct((m, 1), jnp.float32),
    )(c0, c1, w_row)


def kernel(x, edge_index, edge_weight, W0, W1, W_lin):
    row = edge_index[0].astype(jnp.int32)
    col = edge_index[1].astype(jnp.int32)
    w = edge_weight.astype(jnp.float32)

    pad = EPAD - E
    prow = jnp.zeros((pad,), jnp.int32)
    pcol = (jnp.arange(pad, dtype=jnp.int32) * 7) % N
    pw = jnp.zeros((pad,), jnp.float32)
    row3 = jnp.concatenate([row, prow]).reshape(NW, NWIN, WIN)
    col3 = jnp.concatenate([col, pcol]).reshape(NW, NWIN, WIN)
    w3 = jnp.concatenate([w, pw]).reshape(NW, NWIN, WIN)

    deg_parts = _sc_degree(col3, w3)[:, :N]
    dis = _tc_dis(deg_parts)

    xw0 = _tc_matmul(x, W0)
    p0 = _sc_aggregate(xw0, row3, col3, w3, dis)

    hw1 = _tc_relu_matmul(p0[0, :N], p0[1, :N], W1)
    p1 = _sc_aggregate(hw1, row3, col3, w3, dis)

    c0 = p1[0, :N:GROUP]
    c1 = p1[1, :N:GROUP]
    w_row = W_lin.reshape(1, H)
    return _tc_head(c0, c1, w_row)


# speedup vs baseline: 6.3767x; 1.0035x over previous
import functools

import jax
import jax.numpy as jnp
from jax import lax
from jax.experimental import pallas as pl
from jax.experimental.pallas import tpu as pltpu
from jax.experimental.pallas import tpu_sc as plsc

N = 10000
E = 320000
D = 128
H = 128
GROUP = 10

NC = 2
NS = 16
NW = NC * NS
WIN = 128
NWIN = 80
CHW = 16
NCH = NWIN // CHW
EPW = WIN * NWIN
EPAD = NW * EPW
NP = 10240
ROWS_PER_TILE = NP // NS

_mesh = functools.partial(
    plsc.VectorSubcoreMesh, core_axis_name="c", subcore_axis_name="s",
    num_cores=NC, num_subcores=NS)


def _deg_body(col_hbm, w_hbm, deg_out, col_v, w_v, zero_v, deg_sp):
    cid = lax.axis_index("c")
    sid = lax.axis_index("s")
    wid = cid * NS + sid

    pltpu.sync_copy(col_hbm.at[wid], col_v)
    pltpu.sync_copy(w_hbm.at[wid], w_v)

    @pl.when(sid == 0)
    def _():
        z16 = jnp.zeros((16,), jnp.float32)

        def zb(i, c):
            zero_v[pl.ds(i * 16, 16)] = z16
            return c
        lax.fori_loop(0, NP // 16, zb, 0)
        pltpu.sync_copy(zero_v, deg_sp)

    plsc.subcore_barrier()

    def win(wi, c):
        pltpu.sync_copy(w_v.at[wi], deg_sp.at[col_v.at[wi]], add=True)
        return c
    lax.fori_loop(0, NWIN, win, 0)

    plsc.subcore_barrier()

    @pl.when(sid == 0)
    def _():
        pltpu.sync_copy(deg_sp, deg_out.at[cid])


def _sc_degree(col3, w3):
    k = pl.kernel(
        _deg_body,
        out_type=jax.ShapeDtypeStruct((NC, NP), jnp.float32),
        mesh=_mesh(),
        compiler_params=pltpu.CompilerParams(needs_layout_passes=False),
        scratch_types=[
            pltpu.VMEM((NWIN, WIN), jnp.int32),
            pltpu.VMEM((NWIN, WIN), jnp.float32),
            pltpu.VMEM((NP,), jnp.float32),
            pltpu.VMEM_SHARED((NP,), jnp.float32),
        ],
    )
    return k(col3, w3)


def _agg_body(xw_hbm, row_hbm, col_hbm, w_hbm, dis_hbm, out_hbm,
              row_v, col_v, w_v, dis_v, msg_v, acc_sp, sem):
    cid = lax.axis_index("c")
    sid = lax.axis_index("s")
    wid = cid * NS + sid

    pltpu.sync_copy(dis_hbm, dis_v)

    z16 = jnp.zeros((16,), jnp.float32)

    @plsc.parallel_loop(0, WIN, unroll=4)
    def zb(i):
        for g in range(8):
            msg_v[i, pl.ds(g * 16, 16)] = z16
    for j in range(ROWS_PER_TILE // WIN):
        pltpu.sync_copy(msg_v, acc_sp.at[pl.ds(sid * ROWS_PER_TILE + j * WIN, WIN)])

    plsc.subcore_barrier()

    def chunk(ci, c0):
        pltpu.sync_copy(row_hbm.at[wid, pl.ds(ci * CHW, CHW)], row_v)
        pltpu.sync_copy(col_hbm.at[wid, pl.ds(ci * CHW, CHW)], col_v)
        pltpu.sync_copy(w_hbm.at[wid, pl.ds(ci * CHW, CHW)], w_v)

        def win(wi, c):
            pltpu.async_copy(xw_hbm.at[row_v.at[wi]], msg_v, sem).wait()

            @plsc.parallel_loop(0, 8, unroll=2)
            def scale(g):
                r16 = row_v[wi, pl.ds(g * 16, 16)]
                c16 = col_v[wi, pl.ds(g * 16, 16)]
                w16 = w_v[wi, pl.ds(g * 16, 16)]
                nm16 = plsc.load_gather(dis_v, [r16]) * w16 * plsc.load_gather(dis_v, [c16])
                for l in range(16):
                    sv = jnp.full((16,), nm16[l], jnp.float32)
                    e = g * 16 + l
                    for ch in range(8):
                        msg_v[e, pl.ds(ch * 16, 16)] = msg_v[e, pl.ds(ch * 16, 16)] * sv

            pltpu.sync_copy(msg_v, acc_sp.at[col_v.at[wi]], add=True)
            return c
        lax.fori_loop(0, CHW, win, 0)
        return c0
    lax.fori_loop(0, NCH, chunk, 0)

    plsc.subcore_barrier()
    pltpu.sync_copy(acc_sp.at[pl.ds(sid * ROWS_PER_TILE, ROWS_PER_TILE)],
                    out_hbm.at[cid, pl.ds(sid * ROWS_PER_TILE, ROWS_PER_TILE)])


def _sc_aggregate(xw, row3, col3, w3, dis):
    k = pl.kernel(
        _agg_body,
        out_type=jax.ShapeDtypeStruct((NC, NP, H), jnp.float32),
        mesh=_mesh(),
        compiler_params=pltpu.CompilerParams(needs_layout_passes=False),
        scratch_types=[
            pltpu.VMEM((CHW, WIN), jnp.int32),
            pltpu.VMEM((CHW, WIN), jnp.int32),
            pltpu.VMEM((CHW, WIN), jnp.float32),
            pltpu.VMEM((N,), jnp.float32),
            pltpu.VMEM((WIN, H), jnp.float32),
            pltpu.VMEM_SHARED((NP, H), jnp.float32),
            pltpu.SemaphoreType.DMA,
        ],
    )
    return k(xw, row3, col3, w3, dis)


def _dis_body(dp_ref, out_ref):
    deg = dp_ref[0] + dp_ref[1]
    out_ref[...] = jnp.where(
        deg > 0, lax.rsqrt(jnp.maximum(deg, 1e-12)), 0.0)


def _tc_dis(deg_parts):
    npad = 80 * 128
    dp = jnp.pad(deg_parts, ((0, 0), (0, npad - N))).reshape(NC, 80, 128)
    out = pl.pallas_call(
        _dis_body,
        out_shape=jax.ShapeDtypeStruct((80, 128), jnp.float32),
    )(dp)
    return out.reshape(npad)[:N]


def _mm_body(x_ref, w_ref, o_ref):
    o_ref[...] = jnp.dot(x_ref[...], w_ref[...],
                         preferred_element_type=jnp.float32)


def _tc_matmul(x, w):
    m = x.shape[0]
    bm = 400
    return pl.pallas_call(
        _mm_body,
        grid=(m // bm,),
        in_specs=[pl.BlockSpec((bm, x.shape[1]), lambda i: (i, 0)),
                  pl.BlockSpec(w.shape, lambda i: (0, 0))],
        out_specs=pl.BlockSpec((bm, w.shape[1]), lambda i: (i, 0)),
        out_shape=jax.ShapeDtypeStruct((m, w.shape[1]), jnp.float32),
    )(x, w)


def _mid_body(a_ref, b_ref, w_ref, o_ref):
    h = jnp.maximum(a_ref[...] + b_ref[...], 0.0)
    o_ref[...] = jnp.dot(h, w_ref[...], preferred_element_type=jnp.float32)


def _tc_relu_matmul(a, b, w):
    m = a.shape[0]
    bm = 400
    return pl.pallas_call(
        _mid_body,
        grid=(m // bm,),
        in_specs=[pl.BlockSpec((bm, a.shape[1]), lambda i: (i, 0)),
                  pl.BlockSpec((bm, a.shape[1]), lambda i: (i, 0)),
                  pl.BlockSpec(w.shape, lambda i: (0, 0))],
        out_specs=pl.BlockSpec((bm, w.shape[1]), lambda i: (i, 0)),
        out_shape=jax.ShapeDtypeStruct((m, w.shape[1]), jnp.float32),
    )(a, b, w)


def _head_body(a_ref, b_ref, w_ref, o_ref):
    h = jnp.maximum(a_ref[...] + b_ref[...], 0.0)
    p = jnp.sum(h * w_ref[...], axis=1, keepdims=True)
    o_ref[...] = jnp.maximum(p, 0.0)


def _tc_head(c0, c1, w_row):
    m = c0.shape[0]
    return pl.pallas_call(
        _head_body,
        out_shape=jax.ShapeDtypeStruct((m, 1), jnp.float32),
    )(c0, c1, w_row)


def kernel(x, edge_index, edge_weight, W0, W1, W_lin):
    row = edge_index[0].astype(jnp.int32)
    col = edge_index[1].astype(jnp.int32)
    w = edge_weight.astype(jnp.float32)

    pad = EPAD - E
    prow = jnp.zeros((pad,), jnp.int32)
    pcol = (jnp.arange(pad, dtype=jnp.int32) * 7) % N
    pw = jnp.zeros((pad,), jnp.float32)
    row3 = jnp.concatenate([row, prow]).reshape(NW, NWIN, WIN)
    col3 = jnp.concatenate([col, pcol]).reshape(NW, NWIN, WIN)
    w3 = jnp.concatenate([w, pw]).reshape(NW, NWIN, WIN)

    deg_parts = _sc_degree(col3, w3)[:, :N]
    dis = _tc_dis(deg_parts)

    xw0 = _tc_matmul(x, W0)
    p0 = _sc_aggregate(xw0, row3, col3, w3, dis)

    hw1 = _tc_relu_matmul(p0[0, :N], p0[1, :N], W1)
    p1 = _sc_aggregate(hw1, row3, col3, w3, dis)

    c0 = p1[0, :N:GROUP]
    c1 = p1[1, :N:GROUP]
    w_row = W_lin.reshape(1, H)
    return _tc_head(c0, c1, w_row)


# speedup vs baseline: 9.2362x; 1.4484x over previous
import functools

import jax
import jax.numpy as jnp
from jax import lax
from jax.experimental import pallas as pl
from jax.experimental.pallas import tpu as pltpu
from jax.experimental.pallas import tpu_sc as plsc

N = 10000
E = 320000
D = 128
H = 128
GROUP = 10

NC = 2
NS = 16
NW = NC * NS
WIN = 128
NWIN = 80
CHW = 16
NCH = NWIN // CHW
EPW = WIN * NWIN
EPAD = NW * EPW
NP = 10240
ROWS_PER_TILE = NP // NS

_mesh = functools.partial(
    plsc.VectorSubcoreMesh, core_axis_name="c", subcore_axis_name="s",
    num_cores=NC, num_subcores=NS)


def _deg_body(col_hbm, w_hbm, deg_out, col_v, w_v, zero_v, deg_sp):
    cid = lax.axis_index("c")
    sid = lax.axis_index("s")
    wid = cid * NS + sid

    pltpu.sync_copy(col_hbm.at[wid], col_v)
    pltpu.sync_copy(w_hbm.at[wid], w_v)

    @pl.when(sid == 0)
    def _():
        z16 = jnp.zeros((16,), jnp.float32)

        def zb(i, c):
            zero_v[pl.ds(i * 16, 16)] = z16
            return c
        lax.fori_loop(0, NP // 16, zb, 0)
        pltpu.sync_copy(zero_v, deg_sp)

    plsc.subcore_barrier()

    def win(wi, c):
        pltpu.sync_copy(w_v.at[wi], deg_sp.at[col_v.at[wi]], add=True)
        return c
    lax.fori_loop(0, NWIN, win, 0)

    plsc.subcore_barrier()

    @pl.when(sid == 0)
    def _():
        pltpu.sync_copy(deg_sp, deg_out.at[cid])


def _sc_degree(col3, w3):
    k = pl.kernel(
        _deg_body,
        out_type=jax.ShapeDtypeStruct((NC, NP), jnp.float32),
        mesh=_mesh(),
        compiler_params=pltpu.CompilerParams(needs_layout_passes=False),
        scratch_types=[
            pltpu.VMEM((NWIN, WIN), jnp.int32),
            pltpu.VMEM((NWIN, WIN), jnp.float32),
            pltpu.VMEM((NP,), jnp.float32),
            pltpu.VMEM_SHARED((NP,), jnp.float32),
        ],
    )
    return k(col3, w3)


def _agg_body(xw_hbm, row_hbm, col_hbm, w_hbm, dis_hbm, out_hbm,
              row_v, col_v, w_v, dis_v, msg_v, acc_sp, sem):
    cid = lax.axis_index("c")
    sid = lax.axis_index("s")
    wid = cid * NS + sid

    pltpu.sync_copy(dis_hbm, dis_v)

    z16 = jnp.zeros((16,), jnp.float32)

    @plsc.parallel_loop(0, WIN, unroll=4)
    def zb(i):
        for g in range(8):
            msg_v[i, pl.ds(g * 16, 16)] = z16
    for j in range(ROWS_PER_TILE // WIN):
        pltpu.sync_copy(msg_v, acc_sp.at[pl.ds(sid * ROWS_PER_TILE + j * WIN, WIN)])

    plsc.subcore_barrier()

    def chunk(ci, c0):
        pltpu.sync_copy(row_hbm.at[wid, pl.ds(ci * CHW, CHW)], row_v)
        pltpu.sync_copy(col_hbm.at[wid, pl.ds(ci * CHW, CHW)], col_v)
        pltpu.sync_copy(w_hbm.at[wid, pl.ds(ci * CHW, CHW)], w_v)

        def win(wi, c):
            pltpu.async_copy(xw_hbm.at[row_v.at[wi]], msg_v, sem).wait()

            @plsc.parallel_loop(0, 8, unroll=2)
            def scale(g):
                r16 = row_v[wi, pl.ds(g * 16, 16)]
                c16 = col_v[wi, pl.ds(g * 16, 16)]
                w16 = w_v[wi, pl.ds(g * 16, 16)]
                nm16 = plsc.load_gather(dis_v, [r16]) * w16 * plsc.load_gather(dis_v, [c16])
                for l in range(16):
                    sv = jnp.full((16,), nm16[l], jnp.float32)
                    e = g * 16 + l
                    for ch in range(8):
                        msg_v[e, pl.ds(ch * 16, 16)] = msg_v[e, pl.ds(ch * 16, 16)] * sv

            pltpu.sync_copy(msg_v, acc_sp.at[col_v.at[wi]], add=True)
            return c
        lax.fori_loop(0, CHW, win, 0)
        return c0
    lax.fori_loop(0, NCH, chunk, 0)

    plsc.subcore_barrier()
    pltpu.sync_copy(acc_sp.at[pl.ds(sid * ROWS_PER_TILE, ROWS_PER_TILE)],
                    out_hbm.at[cid, pl.ds(sid * ROWS_PER_TILE, ROWS_PER_TILE)])


def _sc_aggregate(xw, row3, col3, w3, dis):
    k = pl.kernel(
        _agg_body,
        out_type=jax.ShapeDtypeStruct((NC, NP, H), jnp.float32),
        mesh=_mesh(),
        compiler_params=pltpu.CompilerParams(needs_layout_passes=False),
        scratch_types=[
            pltpu.VMEM((CHW, WIN), jnp.int32),
            pltpu.VMEM((CHW, WIN), jnp.int32),
            pltpu.VMEM((CHW, WIN), jnp.float32),
            pltpu.VMEM((N,), jnp.float32),
            pltpu.VMEM((WIN, H), jnp.float32),
            pltpu.VMEM_SHARED((NP, H), jnp.float32),
            pltpu.SemaphoreType.DMA,
        ],
    )
    return k(xw, row3, col3, w3, dis)


NCEN = 1024


def _aggc_body(xw_hbm, row_hbm, col_hbm, w_hbm, dis_hbm, out_hbm,
               row_v, col_v, w_v, rowm_v, colm_v, dis_v, msg_v, acc_sp, sem):
    cid = lax.axis_index("c")
    sid = lax.axis_index("s")
    wid = cid * NS + sid

    pltpu.sync_copy(dis_hbm, dis_v)

    z16 = jnp.zeros((16,), jnp.float32)

    @plsc.parallel_loop(0, 64, unroll=4)
    def zb(i):
        for g in range(8):
            msg_v[i, pl.ds(g * 16, 16)] = z16
    pltpu.sync_copy(msg_v.at[pl.ds(0, 64)], acc_sp.at[pl.ds(sid * 64, 64)])

    plsc.subcore_barrier()

    def chunk(ci, c0):
        pltpu.sync_copy(row_hbm.at[wid, pl.ds(ci * CHW, CHW)], row_v)
        pltpu.sync_copy(col_hbm.at[wid, pl.ds(ci * CHW, CHW)], col_v)
        pltpu.sync_copy(w_hbm.at[wid, pl.ds(ci * CHW, CHW)], w_v)

        def win(wi, c):
            @plsc.parallel_loop(0, 8)
            def mk(g):
                c16 = col_v[wi, pl.ds(g * 16, 16)]
                r16 = row_v[wi, pl.ds(g * 16, 16)]
                q = c16 // GROUP
                is_c = c16 == q * GROUP
                colm_v[pl.ds(g * 16, 16)] = jnp.where(is_c, q, -1)
                rowm_v[pl.ds(g * 16, 16)] = jnp.where(is_c, r16, -1)

            pltpu.async_copy(xw_hbm.at[plsc.Indices(rowm_v, ignored_value=-1)],
                             msg_v, sem).wait()

            def scale(g, c2):
                r16 = row_v[wi, pl.ds(g * 16, 16)]
                c16 = colm_v[pl.ds(g * 16, 16)]
                w16 = w_v[wi, pl.ds(g * 16, 16)]
                nm16 = plsc.load_gather(dis_v, [r16]) * w16 * plsc.load_gather(
                    dis_v, [col_v[wi, pl.ds(g * 16, 16)]])
                for l in range(16):
                    @pl.when(c16[l] >= 0)
                    def _():
                        sv = jnp.full((16,), nm16[l], jnp.float32)
                        e = g * 16 + l
                        for ch in range(8):
                            msg_v[e, pl.ds(ch * 16, 16)] = (
                                msg_v[e, pl.ds(ch * 16, 16)] * sv)
                return c2
            lax.fori_loop(0, 8, scale, 0)

            pltpu.sync_copy(msg_v,
                            acc_sp.at[plsc.Indices(colm_v, ignored_value=-1)],
                            add=True)
            return c
        lax.fori_loop(0, CHW, win, 0)
        return c0
    lax.fori_loop(0, NCH, chunk, 0)

    plsc.subcore_barrier()
    pltpu.sync_copy(acc_sp.at[pl.ds(sid * 64, 64)],
                    out_hbm.at[cid, pl.ds(sid * 64, 64)])


def _sc_aggregate_centers(xw, row3, col3, w3, dis):
    k = pl.kernel(
        _aggc_body,
        out_type=jax.ShapeDtypeStruct((NC, NCEN, H), jnp.float32),
        mesh=_mesh(),
        compiler_params=pltpu.CompilerParams(needs_layout_passes=False),
        scratch_types=[
            pltpu.VMEM((CHW, WIN), jnp.int32),
            pltpu.VMEM((CHW, WIN), jnp.int32),
            pltpu.VMEM((CHW, WIN), jnp.float32),
            pltpu.VMEM((WIN,), jnp.int32),
            pltpu.VMEM((WIN,), jnp.int32),
            pltpu.VMEM((N,), jnp.float32),
            pltpu.VMEM((WIN, H), jnp.float32),
            pltpu.VMEM_SHARED((NCEN, H), jnp.float32),
            pltpu.SemaphoreType.DMA,
        ],
    )
    return k(xw, row3, col3, w3, dis)


def _dis_body(dp_ref, out_ref):
    deg = dp_ref[0] + dp_ref[1]
    out_ref[...] = jnp.where(
        deg > 0, lax.rsqrt(jnp.maximum(deg, 1e-12)), 0.0)


def _tc_dis(deg_parts):
    npad = 80 * 128
    dp = jnp.pad(deg_parts, ((0, 0), (0, npad - N))).reshape(NC, 80, 128)
    out = pl.pallas_call(
        _dis_body,
        out_shape=jax.ShapeDtypeStruct((80, 128), jnp.float32),
    )(dp)
    return out.reshape(npad)[:N]


def _mm_body(x_ref, w_ref, o_ref):
    o_ref[...] = jnp.dot(x_ref[...], w_ref[...],
                         preferred_element_type=jnp.float32)


def _tc_matmul(x, w):
    m = x.shape[0]
    bm = 400
    return pl.pallas_call(
        _mm_body,
        grid=(m // bm,),
        in_specs=[pl.BlockSpec((bm, x.shape[1]), lambda i: (i, 0)),
                  pl.BlockSpec(w.shape, lambda i: (0, 0))],
        out_specs=pl.BlockSpec((bm, w.shape[1]), lambda i: (i, 0)),
        out_shape=jax.ShapeDtypeStruct((m, w.shape[1]), jnp.float32),
    )(x, w)


def _mid_body(a_ref, b_ref, w_ref, o_ref):
    h = jnp.maximum(a_ref[...] + b_ref[...], 0.0)
    o_ref[...] = jnp.dot(h, w_ref[...], preferred_element_type=jnp.float32)


def _tc_relu_matmul(a, b, w):
    m = a.shape[0]
    bm = 400
    return pl.pallas_call(
        _mid_body,
        grid=(m // bm,),
        in_specs=[pl.BlockSpec((bm, a.shape[1]), lambda i: (i, 0)),
                  pl.BlockSpec((bm, a.shape[1]), lambda i: (i, 0)),
                  pl.BlockSpec(w.shape, lambda i: (0, 0))],
        out_specs=pl.BlockSpec((bm, w.shape[1]), lambda i: (i, 0)),
        out_shape=jax.ShapeDtypeStruct((m, w.shape[1]), jnp.float32),
    )(a, b, w)


def _head_body(a_ref, b_ref, w_ref, o_ref):
    h = jnp.maximum(a_ref[...] + b_ref[...], 0.0)
    p = jnp.sum(h * w_ref[...], axis=1, keepdims=True)
    o_ref[...] = jnp.maximum(p, 0.0)


def _tc_head(c0, c1, w_row):
    m = c0.shape[0]
    return pl.pallas_call(
        _head_body,
        out_shape=jax.ShapeDtypeStruct((m, 1), jnp.float32),
    )(c0, c1, w_row)


def kernel(x, edge_index, edge_weight, W0, W1, W_lin):
    row = edge_index[0].astype(jnp.int32)
    col = edge_index[1].astype(jnp.int32)
    w = edge_weight.astype(jnp.float32)

    pad = EPAD - E
    prow = jnp.zeros((pad,), jnp.int32)
    pcol = (jnp.arange(pad, dtype=jnp.int32) * 7) % N
    pw = jnp.zeros((pad,), jnp.float32)
    row3 = jnp.concatenate([row, prow]).reshape(NW, NWIN, WIN)
    col3 = jnp.concatenate([col, pcol]).reshape(NW, NWIN, WIN)
    w3 = jnp.concatenate([w, pw]).reshape(NW, NWIN, WIN)

    deg_parts = _sc_degree(col3, w3)[:, :N]
    dis = _tc_dis(deg_parts)

    xw0 = _tc_matmul(x, W0)
    p0 = _sc_aggregate(xw0, row3, col3, w3, dis)

    hw1 = _tc_relu_matmul(p0[0, :N], p0[1, :N], W1)
    p1 = _sc_aggregate_centers(hw1, row3, col3, w3, dis)

    c0 = p1[0, :N // GROUP]
    c1 = p1[1, :N // GROUP]
    w_row = W_lin.reshape(1, H)
    return _tc_head(c0, c1, w_row)
